# baseline (speedup 1.0000x reference)
import functools

import jax
import jax.numpy as jnp
from jax import lax
from jax.experimental import pallas as pl
from jax.experimental.pallas import tpu as pltpu
from jax.experimental.pallas import tpu_sc as plsc

_VOCAB = 100000
_EMBED = 64
_CONTEXT = 200
_HIDDEN = 128
_FLAT = _CONTEXT * _EMBED

_TILE = 10240
_NSTREAM = 1
_QUANT = _TILE * _NSTREAM
_NV = (_VOCAB + _QUANT - 1) // _QUANT
_NCHUNK = _NV * _NSTREAM
_PADDED = _NCHUNK * _TILE

_NC = 2
_NS = 16
_ROWS_PER_W = 8
_NW_ACTIVE = _CONTEXT // _ROWS_PER_W


def _sc_gather(embt_hbm, idx_hbm, out_hbm, idx_v, land, flat_v, sem):
    wid = lax.axis_index("s") * _NC + lax.axis_index("c")

    @pl.when(wid < _NW_ACTIVE)
    def _():
        base = wid * _ROWS_PER_W
        pltpu.sync_copy(idx_hbm.at[pl.ds(base, _ROWS_PER_W)],
                        idx_v.at[pl.ds(0, _ROWS_PER_W)])
        vec = idx_v[...]
        lane = lax.iota(jnp.int32, 16)
        cols = []
        copies = []
        for k in range(_ROWS_PER_W):
            col = jnp.sum(jnp.where(lane == k, vec, 0), axis=0)
            col0 = pl.multiple_of((col // 128) * 128, 128)
            cols.append(col - col0)
            copies.append(pltpu.async_copy(
                embt_hbm.at[:, pl.ds(col0, 128)], land.at[k], sem))
        for c in copies:
            c.wait()
        for k in range(_ROWS_PER_W):
            lane_in_tile = jnp.full((16,), cols[k], dtype=jnp.int32)
            for c in range(_EMBED // 16):
                rows16 = lane + c * 16
                flat_v[pl.ds(k * _EMBED + c * 16, 16)] = plsc.load_gather(
                    land.at[k], [rows16, lane_in_tile])
        pltpu.sync_copy(flat_v, out_hbm.at[pl.ds(base * _EMBED,
                                                 _ROWS_PER_W * _EMBED)])


@functools.cache
def _sc_gather_call():
    return pl.kernel(
        _sc_gather,
        out_type=jax.ShapeDtypeStruct((_FLAT,), jnp.float32),
        mesh=plsc.VectorSubcoreMesh(core_axis_name="c", subcore_axis_name="s"),
        scratch_types=[
            pltpu.VMEM((16,), jnp.int32),
            pltpu.VMEM((_ROWS_PER_W, _EMBED, 128), jnp.float32),
            pltpu.VMEM((_ROWS_PER_W * _EMBED,), jnp.float32),
            pltpu.SemaphoreType.DMA,
        ],
        compiler_params=pltpu.CompilerParams(
            needs_layout_passes=False,
        ),
    )


def _tc_mlp(embeds_ref, w1_ref, b1_ref, *rest):
    w2_refs = rest[:_NSTREAM]
    b2_refs = rest[_NSTREAM:2 * _NSTREAM]
    out_ref = rest[2 * _NSTREAM]
    h_ref, logits_ref, m_ref = rest[2 * _NSTREAM + 1:]
    j = pl.program_id(0)

    @pl.when(j == 0)
    def _():
        e = embeds_ref[...].reshape(1, _FLAT)
        h = lax.dot_general(e, w1_ref[...],
                            (((1,), (1,)), ((), ())),
                            preferred_element_type=jnp.float32) + b1_ref[...]
        h_ref[...] = jnp.maximum(h, 0.0)
        m_ref[0] = -jnp.inf

    ts = []
    for q in range(_NSTREAM):
        ts.append(lax.dot_general(h_ref[...], w2_refs[q][...],
                                  (((1,), (1,)), ((), ())),
                                  preferred_element_type=jnp.float32)
                  + b2_refs[q][...].reshape(1, _TILE))
    if_last = j == _NV - 1

    @pl.when(jnp.logical_not(if_last))
    def _():
        m = m_ref[0]
        for q in range(_NSTREAM):
            logits_ref[:, pl.ds((j * _NSTREAM + q) * _TILE, _TILE)] = ts[q]
            m = jnp.maximum(m, jnp.max(ts[q]))
        m_ref[0] = m

    @pl.when(if_last)
    def _():
        m = m_ref[0]
        for q in range(_NSTREAM):
            col = ((j * _NSTREAM + q) * _TILE
                   + lax.broadcasted_iota(jnp.int32, (1, _TILE), 1))
            tm = jnp.where(col < _VOCAB, ts[q], -jnp.inf)
            logits_ref[:, pl.ds((j * _NSTREAM + q) * _TILE, _TILE)] = tm
            m = jnp.maximum(m, jnp.max(tm))

        def body(i, s):
            chunk = logits_ref[:, pl.ds(i * _TILE, _TILE)]
            return s + jnp.sum(jnp.exp(chunk - m))

        s = lax.fori_loop(0, _NCHUNK, body, 0.0)
        lse = m + jnp.log(s)

        def body2(i, carry):
            out_ref[:, pl.ds(i * _TILE, _TILE)] = (
                logits_ref[:, pl.ds(i * _TILE, _TILE)] - lse)
            return carry

        nfull = _VOCAB // _TILE
        lax.fori_loop(0, nfull, body2, 0)
        tail = _VOCAB - nfull * _TILE
        out_ref[:, pl.ds(nfull * _TILE, tail)] = (
            logits_ref[:, pl.ds(nfull * _TILE, tail)] - lse)


@functools.cache
def _tc_mlp_call(interpret=False):
    last_valid = (_VOCAB - 1) // _TILE

    w2_specs = [
        pl.BlockSpec((_TILE, _HIDDEN), functools.partial(
            lambda q, j: (jnp.minimum(j * _NSTREAM + q, last_valid), 0), q))
        for q in range(_NSTREAM)
    ]
    b2_specs = [
        pl.BlockSpec((_TILE,), functools.partial(
            lambda q, j: (jnp.minimum(j * _NSTREAM + q, last_valid),), q))
        for q in range(_NSTREAM)
    ]
    return pl.pallas_call(
        _tc_mlp,
        grid=(_NV,),
        in_specs=[
            pl.BlockSpec((_FLAT,), lambda j: (0,)),
            pl.BlockSpec((_HIDDEN, _FLAT), lambda j: (0, 0)),
            pl.BlockSpec((1, _HIDDEN), lambda j: (0, 0)),
            *w2_specs,
            *b2_specs,
        ],
        out_specs=pl.BlockSpec((1, _VOCAB), lambda j: (0, 0)),
        out_shape=jax.ShapeDtypeStruct((1, _VOCAB), jnp.float32),
        scratch_shapes=[
            pltpu.VMEM((1, _HIDDEN), jnp.float32),
            pltpu.VMEM((1, _PADDED), jnp.float32),
            pltpu.SMEM((1,), jnp.float32),
        ],
        compiler_params=pltpu.CompilerParams(
            dimension_semantics=("arbitrary",),
        ),
        interpret=interpret,
    )


@jax.jit
def kernel(inputs, emb, W1, b1, W2, b2):
    embeds = _sc_gather_call()(emb.T, inputs)
    return _tc_mlp_call()(embeds, W1, b1.reshape(1, _HIDDEN),
                          *([W2] * _NSTREAM), *([b2] * _NSTREAM))


# speedup vs baseline: 1.0593x; 1.0593x over previous
import functools

import jax
import jax.numpy as jnp
from jax import lax
from jax.experimental import pallas as pl
from jax.experimental.pallas import tpu as pltpu
from jax.experimental.pallas import tpu_sc as plsc

_VOCAB = 100000
_EMBED = 64
_CONTEXT = 200
_HIDDEN = 128
_FLAT = _CONTEXT * _EMBED

_TILE = 25600
_NSTREAM = 1
_QUANT = _TILE * _NSTREAM
_NV = (_VOCAB + _QUANT - 1) // _QUANT
_NCHUNK = _NV * _NSTREAM
_PADDED = _NCHUNK * _TILE

_NC = 2
_NS = 16
_ROWS_PER_W = 8
_NW_ACTIVE = _CONTEXT // _ROWS_PER_W


def _sc_gather(embt_hbm, idx_hbm, out_hbm, idx_v, land, flat_v, sem):
    wid = lax.axis_index("s") * _NC + lax.axis_index("c")

    @pl.when(wid < _NW_ACTIVE)
    def _():
        base = wid * _ROWS_PER_W
        pltpu.sync_copy(idx_hbm.at[pl.ds(base, _ROWS_PER_W)],
                        idx_v.at[pl.ds(0, _ROWS_PER_W)])
        vec = idx_v[...]
        lane = lax.iota(jnp.int32, 16)
        cols = []
        copies = []
        for k in range(_ROWS_PER_W):
            col = jnp.sum(jnp.where(lane == k, vec, 0), axis=0)
            col0 = pl.multiple_of((col // 128) * 128, 128)
            cols.append(col - col0)
            copies.append(pltpu.async_copy(
                embt_hbm.at[:, pl.ds(col0, 128)], land.at[k], sem))
        for c in copies:
            c.wait()
        for k in range(_ROWS_PER_W):
            lane_in_tile = jnp.full((16,), cols[k], dtype=jnp.int32)
            for c in range(_EMBED // 16):
                rows16 = lane + c * 16
                flat_v[pl.ds(k * _EMBED + c * 16, 16)] = plsc.load_gather(
                    land.at[k], [rows16, lane_in_tile])
        pltpu.sync_copy(flat_v, out_hbm.at[pl.ds(base * _EMBED,
                                                 _ROWS_PER_W * _EMBED)])


@functools.cache
def _sc_gather_call():
    return pl.kernel(
        _sc_gather,
        out_type=jax.ShapeDtypeStruct((_FLAT,), jnp.float32),
        mesh=plsc.VectorSubcoreMesh(core_axis_name="c", subcore_axis_name="s"),
        scratch_types=[
            pltpu.VMEM((16,), jnp.int32),
            pltpu.VMEM((_ROWS_PER_W, _EMBED, 128), jnp.float32),
            pltpu.VMEM((_ROWS_PER_W * _EMBED,), jnp.float32),
            pltpu.SemaphoreType.DMA,
        ],
        compiler_params=pltpu.CompilerParams(
            needs_layout_passes=False,
        ),
    )


def _tc_mlp(embeds_ref, w1_ref, b1_ref, *rest):
    w2_refs = rest[:_NSTREAM]
    b2_refs = rest[_NSTREAM:2 * _NSTREAM]
    out_ref = rest[2 * _NSTREAM]
    h_ref, logits_ref, m_ref = rest[2 * _NSTREAM + 1:]
    j = pl.program_id(0)

    @pl.when(j == 0)
    def _():
        e = embeds_ref[...].reshape(1, _FLAT)
        h = lax.dot_general(e, w1_ref[...],
                            (((1,), (1,)), ((), ())),
                            preferred_element_type=jnp.float32) + b1_ref[...]
        h_ref[...] = jnp.maximum(h, 0.0)
        m_ref[0] = -jnp.inf

    ts = []
    for q in range(_NSTREAM):
        ts.append(lax.dot_general(h_ref[...], w2_refs[q][...],
                                  (((1,), (1,)), ((), ())),
                                  preferred_element_type=jnp.float32)
                  + b2_refs[q][...].reshape(1, _TILE))
    if_last = j == _NV - 1

    @pl.when(jnp.logical_not(if_last))
    def _():
        m = m_ref[0]
        for q in range(_NSTREAM):
            logits_ref[:, pl.ds((j * _NSTREAM + q) * _TILE, _TILE)] = ts[q]
            m = jnp.maximum(m, jnp.max(ts[q]))
        m_ref[0] = m

    @pl.when(if_last)
    def _():
        m = m_ref[0]
        for q in range(_NSTREAM):
            col = ((j * _NSTREAM + q) * _TILE
                   + lax.broadcasted_iota(jnp.int32, (1, _TILE), 1))
            tm = jnp.where(col < _VOCAB, ts[q], -jnp.inf)
            logits_ref[:, pl.ds((j * _NSTREAM + q) * _TILE, _TILE)] = tm
            m = jnp.maximum(m, jnp.max(tm))

        def body(i, s):
            chunk = logits_ref[:, pl.ds(i * _TILE, _TILE)]
            return s + jnp.sum(jnp.exp(chunk - m))

        s = lax.fori_loop(0, _NCHUNK, body, 0.0)
        lse = m + jnp.log(s)

        def body2(i, carry):
            out_ref[:, pl.ds(i * _TILE, _TILE)] = (
                logits_ref[:, pl.ds(i * _TILE, _TILE)] - lse)
            return carry

        nfull = _VOCAB // _TILE
        lax.fori_loop(0, nfull, body2, 0)
        tail = _VOCAB - nfull * _TILE
        out_ref[:, pl.ds(nfull * _TILE, tail)] = (
            logits_ref[:, pl.ds(nfull * _TILE, tail)] - lse)


@functools.cache
def _tc_mlp_call(interpret=False):
    last_valid = (_VOCAB - 1) // _TILE

    w2_specs = [
        pl.BlockSpec((_TILE, _HIDDEN), functools.partial(
            lambda q, j: (jnp.minimum(j * _NSTREAM + q, last_valid), 0), q))
        for q in range(_NSTREAM)
    ]
    b2_specs = [
        pl.BlockSpec((_TILE,), functools.partial(
            lambda q, j: (jnp.minimum(j * _NSTREAM + q, last_valid),), q))
        for q in range(_NSTREAM)
    ]
    return pl.pallas_call(
        _tc_mlp,
        grid=(_NV,),
        in_specs=[
            pl.BlockSpec((_FLAT,), lambda j: (0,)),
            pl.BlockSpec((_HIDDEN, _FLAT), lambda j: (0, 0)),
            pl.BlockSpec((1, _HIDDEN), lambda j: (0, 0)),
            *w2_specs,
            *b2_specs,
        ],
        out_specs=pl.BlockSpec((1, _VOCAB), lambda j: (0, 0)),
        out_shape=jax.ShapeDtypeStruct((1, _VOCAB), jnp.float32),
        scratch_shapes=[
            pltpu.VMEM((1, _HIDDEN), jnp.float32),
            pltpu.VMEM((1, _PADDED), jnp.float32),
            pltpu.SMEM((1,), jnp.float32),
        ],
        compiler_params=pltpu.CompilerParams(
            dimension_semantics=("arbitrary",),
        ),
        interpret=interpret,
    )


@jax.jit
def kernel(inputs, emb, W1, b1, W2, b2):
    embeds = _sc_gather_call()(emb.T, inputs)
    return _tc_mlp_call()(embeds, W1, b1.reshape(1, _HIDDEN),
                          *([W2] * _NSTREAM), *([b2] * _NSTREAM))
